# baseline (speedup 1.0000x reference)
import functools

import jax
import jax.numpy as jnp
from jax.experimental import pallas as pl
from jax.experimental.pallas import tpu as pltpu

N_VARS = 16
N_TGT = 4
INPUT_DIM = 8
HIDDEN = 128
CLS_H = 64
NUM_CLASSES = 10
INV_DEG = 1.0 / 17.0


def _tree_sum(parts):
    while len(parts) > 1:
        parts = [parts[i] + parts[i + 1] for i in range(0, len(parts), 2)]
    return parts[0]


def _fwd_body(f_ref, w1_ref, w2_ref, b2_ref, c1w_ref, c1b_ref,
              c2w_ref, c2b_ref, hw1_ref, hb1_ref, hw2_ref, hb2_ref, out_ref):
    w1 = w1_ref[...]
    w2 = w2_ref[...]
    b2 = b2_ref[...]

    h2 = []
    for v in range(N_VARS):
        h = jnp.maximum(jnp.dot(f_ref[v], w1, preferred_element_type=jnp.float32), 0.0)
        h2.append(jnp.maximum(
            jnp.dot(h, w2, preferred_element_type=jnp.float32) + b2, 0.0))
    s = _tree_sum(list(h2))

    c1w = c1w_ref[...]
    z1 = jnp.dot(s, c1w, preferred_element_type=jnp.float32) + c1b_ref[...]
    x1 = [jnp.maximum(
        jnp.dot(h2[v], c1w, preferred_element_type=jnp.float32) + z1, 0.0)
        for v in range(N_VARS)]
    s1 = _tree_sum(list(x1))

    c2w = c2w_ref[...]
    z2 = jnp.dot(s1, c2w, preferred_element_type=jnp.float32) + c2b_ref[...]
    for i in range(N_TGT):
        x2 = jnp.maximum(
            jnp.dot(x1[i], c2w, preferred_element_type=jnp.float32) + z2, 0.0)
        hh = jnp.maximum(
            jnp.dot(x2, hw1_ref[i], preferred_element_type=jnp.float32) + hb1_ref[i], 0.0)
        out_ref[i] = jnp.dot(hh, hw2_ref[i], preferred_element_type=jnp.float32) + hb2_ref[i]


@functools.partial(jax.jit, static_argnames=("block_b",))
def _run(feats, w1, w2, b2, c1w, c1b, c2w, c2b, hw1, hb1, hw2, hb2,
         block_b=1024):
    b_total = feats.shape[1]
    grid = (b_total // block_b,)

    def full(shape):
        return pl.BlockSpec(shape, lambda i: (0,) * len(shape))

    out = pl.pallas_call(
        _fwd_body,
        grid=grid,
        in_specs=[
            pl.BlockSpec((N_VARS, block_b, INPUT_DIM + 1), lambda i: (0, i, 0)),
            full((INPUT_DIM + 1, HIDDEN)),
            full((HIDDEN, HIDDEN)), full((1, HIDDEN)),
            full((HIDDEN, HIDDEN)), full((1, HIDDEN)),
            full((HIDDEN, HIDDEN)), full((1, HIDDEN)),
            full((N_TGT, HIDDEN, CLS_H)), full((N_TGT, 1, CLS_H)),
            full((N_TGT, CLS_H, NUM_CLASSES)), full((N_TGT, 1, NUM_CLASSES)),
        ],
        out_specs=pl.BlockSpec((N_TGT, block_b, NUM_CLASSES), lambda i: (0, i, 0)),
        out_shape=jax.ShapeDtypeStruct((N_TGT, b_total, NUM_CLASSES), jnp.float32),
        compiler_params=pltpu.CompilerParams(
            dimension_semantics=("parallel",)),
    )(feats, w1, w2, b2, c1w, c1b, c2w, c2b, hw1, hb1, hw2, hb2)
    return out


def kernel(var_0_raw, var_1_raw, var_2_raw, var_3_raw, var_4_raw, var_5_raw,
           var_6_raw, var_7_raw, var_8_raw, var_9_raw, var_10_raw, var_11_raw,
           var_12_raw, var_13_raw, var_14_raw, var_15_raw, params):
    feats = jnp.stack(
        (var_0_raw, var_1_raw, var_2_raw, var_3_raw, var_4_raw, var_5_raw,
         var_6_raw, var_7_raw, var_8_raw, var_9_raw, var_10_raw, var_11_raw,
         var_12_raw, var_13_raw, var_14_raw, var_15_raw), axis=0)
    nv, b_total, _ = feats.shape
    feats = jnp.concatenate(
        [feats, jnp.ones((nv, b_total, 1), jnp.float32)], axis=-1)
    p = params
    w1 = jnp.concatenate([p["enc_W1"], p["enc_b1"].reshape(1, HIDDEN)], axis=0)
    targets = [f"var_{i}" for i in range(N_TGT)]
    hw1 = jnp.stack([p[f"cls_{t}_W1"] for t in targets], axis=0)
    hb1 = jnp.stack([p[f"cls_{t}_b1"].reshape(1, CLS_H) for t in targets], axis=0)
    hw2 = jnp.stack([p[f"cls_{t}_W2"] for t in targets], axis=0)
    hb2 = jnp.stack([p[f"cls_{t}_b2"].reshape(1, NUM_CLASSES) for t in targets], axis=0)
    return _run(
        feats, w1,
        p["enc_W2"], p["enc_b2"].reshape(1, HIDDEN),
        p["conv1_W"] * INV_DEG, p["conv1_b"].reshape(1, HIDDEN),
        p["conv2_W"] * INV_DEG, p["conv2_b"].reshape(1, HIDDEN),
        hw1, hb1, hw2, hb2)


# speedup vs baseline: 1.1023x; 1.1023x over previous
import functools

import jax
import jax.numpy as jnp
from jax.experimental import pallas as pl
from jax.experimental.pallas import tpu as pltpu

N_VARS = 16
N_TGT = 4
INPUT_DIM = 8
HIDDEN = 128
CLS_H = 64
NUM_CLASSES = 10
INV_DEG = 1.0 / 17.0


def _fwd_body(f_ref, w1_ref, b1_ref, w2_ref, b2_ref, c1w_ref, c1b_ref,
              c2w_ref, c2b_ref, hw1_ref, hb1_ref, hw2_ref, hb2_ref, out_ref):
    w1 = w1_ref[...]
    b1 = b1_ref[...]
    w2 = w2_ref[...]
    b2 = b2_ref[...]

    h2 = []
    for v in range(N_VARS):
        f = f_ref[v]
        h = jnp.maximum(jnp.dot(f, w1, preferred_element_type=jnp.float32) + b1, 0.0)
        h = jnp.maximum(jnp.dot(h, w2, preferred_element_type=jnp.float32) + b2, 0.0)
        h2.append(h)
    s = h2[0]
    for v in range(1, N_VARS):
        s = s + h2[v]

    c1w = c1w_ref[...]
    c1b = c1b_ref[...]
    x1 = []
    for v in range(N_VARS):
        t = h2[v] + s
        y = jnp.dot(t, c1w, preferred_element_type=jnp.float32) * INV_DEG + c1b
        x1.append(jnp.maximum(y, 0.0))
    s1 = x1[0]
    for v in range(1, N_VARS):
        s1 = s1 + x1[v]

    c2w = c2w_ref[...]
    c2b = c2b_ref[...]
    for i in range(N_TGT):
        t = x1[i] + s1
        y = jnp.dot(t, c2w, preferred_element_type=jnp.float32) * INV_DEG + c2b
        x2 = jnp.maximum(y, 0.0)
        h = jnp.maximum(
            jnp.dot(x2, hw1_ref[i], preferred_element_type=jnp.float32) + hb1_ref[i], 0.0)
        out_ref[i] = jnp.dot(h, hw2_ref[i], preferred_element_type=jnp.float32) + hb2_ref[i]


@functools.partial(jax.jit, static_argnames=("block_b",))
def _run(feats, w1, b1, w2, b2, c1w, c1b, c2w, c2b, hw1, hb1, hw2, hb2,
         block_b=1024):
    b_total = feats.shape[1]
    grid = (b_total // block_b,)

    def full(shape):
        return pl.BlockSpec(shape, lambda i: (0,) * len(shape))

    out = pl.pallas_call(
        _fwd_body,
        grid=grid,
        in_specs=[
            pl.BlockSpec((N_VARS, block_b, INPUT_DIM), lambda i: (0, i, 0)),
            full((INPUT_DIM, HIDDEN)), full((1, HIDDEN)),
            full((HIDDEN, HIDDEN)), full((1, HIDDEN)),
            full((HIDDEN, HIDDEN)), full((1, HIDDEN)),
            full((HIDDEN, HIDDEN)), full((1, HIDDEN)),
            full((N_TGT, HIDDEN, CLS_H)), full((N_TGT, 1, CLS_H)),
            full((N_TGT, CLS_H, NUM_CLASSES)), full((N_TGT, 1, NUM_CLASSES)),
        ],
        out_specs=pl.BlockSpec((N_TGT, block_b, NUM_CLASSES), lambda i: (0, i, 0)),
        out_shape=jax.ShapeDtypeStruct((N_TGT, b_total, NUM_CLASSES), jnp.float32),
        compiler_params=pltpu.CompilerParams(
            dimension_semantics=("parallel",)),
    )(feats, w1, b1, w2, b2, c1w, c1b, c2w, c2b, hw1, hb1, hw2, hb2)
    return out


def kernel(var_0_raw, var_1_raw, var_2_raw, var_3_raw, var_4_raw, var_5_raw,
           var_6_raw, var_7_raw, var_8_raw, var_9_raw, var_10_raw, var_11_raw,
           var_12_raw, var_13_raw, var_14_raw, var_15_raw, params):
    feats = jnp.stack(
        (var_0_raw, var_1_raw, var_2_raw, var_3_raw, var_4_raw, var_5_raw,
         var_6_raw, var_7_raw, var_8_raw, var_9_raw, var_10_raw, var_11_raw,
         var_12_raw, var_13_raw, var_14_raw, var_15_raw), axis=0)
    p = params
    targets = [f"var_{i}" for i in range(N_TGT)]
    hw1 = jnp.stack([p[f"cls_{t}_W1"] for t in targets], axis=0)
    hb1 = jnp.stack([p[f"cls_{t}_b1"].reshape(1, CLS_H) for t in targets], axis=0)
    hw2 = jnp.stack([p[f"cls_{t}_W2"] for t in targets], axis=0)
    hb2 = jnp.stack([p[f"cls_{t}_b2"].reshape(1, NUM_CLASSES) for t in targets], axis=0)
    return _run(
        feats,
        p["enc_W1"], p["enc_b1"].reshape(1, HIDDEN),
        p["enc_W2"], p["enc_b2"].reshape(1, HIDDEN),
        p["conv1_W"], p["conv1_b"].reshape(1, HIDDEN),
        p["conv2_W"], p["conv2_b"].reshape(1, HIDDEN),
        hw1, hb1, hw2, hb2)


# speedup vs baseline: 1.1039x; 1.0014x over previous
import functools

import jax
import jax.numpy as jnp
from jax.experimental import pallas as pl
from jax.experimental.pallas import tpu as pltpu

N_VARS = 16
N_TGT = 4
INPUT_DIM = 8
HIDDEN = 128
CLS_H = 64
NUM_CLASSES = 10
INV_DEG = 1.0 / 17.0


def _tree_sum(parts):
    while len(parts) > 1:
        parts = [parts[i] + parts[i + 1] for i in range(0, len(parts), 2)]
    return parts[0]


def _fwd_body(f_ref, w1_ref, b1_ref, w2_ref, b2_ref, c1w_ref, c1b_ref,
              c2w_ref, c2b_ref, hw1_ref, hb1_ref, hw2_ref, hb2_ref, out_ref):
    w1 = w1_ref[...]
    b1 = b1_ref[...]
    w2 = w2_ref[...]
    b2 = b2_ref[...]

    h2 = []
    for v in range(N_VARS):
        f = f_ref[v]
        h = jnp.maximum(jnp.dot(f, w1, preferred_element_type=jnp.float32) + b1, 0.0)
        h = jnp.maximum(jnp.dot(h, w2, preferred_element_type=jnp.float32) + b2, 0.0)
        h2.append(h)
    s = _tree_sum(list(h2))

    c1w = c1w_ref[...]
    c1b = c1b_ref[...]
    x1 = []
    for v in range(N_VARS):
        t = h2[v] + s
        y = jnp.dot(t, c1w, preferred_element_type=jnp.float32) * INV_DEG + c1b
        x1.append(jnp.maximum(y, 0.0))
    s1 = _tree_sum(list(x1))

    c2w = c2w_ref[...]
    c2b = c2b_ref[...]
    for i in range(N_TGT):
        t = x1[i] + s1
        y = jnp.dot(t, c2w, preferred_element_type=jnp.float32) * INV_DEG + c2b
        x2 = jnp.maximum(y, 0.0)
        h = jnp.maximum(
            jnp.dot(x2, hw1_ref[i], preferred_element_type=jnp.float32) + hb1_ref[i], 0.0)
        out_ref[i] = jnp.dot(h, hw2_ref[i], preferred_element_type=jnp.float32) + hb2_ref[i]


@functools.partial(jax.jit, static_argnames=("block_b",))
def _run(feats, w1, b1, w2, b2, c1w, c1b, c2w, c2b, hw1, hb1, hw2, hb2,
         block_b=1024):
    b_total = feats.shape[1]
    grid = (b_total // block_b,)

    def full(shape):
        return pl.BlockSpec(shape, lambda i: (0,) * len(shape))

    out = pl.pallas_call(
        _fwd_body,
        grid=grid,
        in_specs=[
            pl.BlockSpec((N_VARS, block_b, INPUT_DIM), lambda i: (0, i, 0)),
            full((INPUT_DIM, HIDDEN)), full((1, HIDDEN)),
            full((HIDDEN, HIDDEN)), full((1, HIDDEN)),
            full((HIDDEN, HIDDEN)), full((1, HIDDEN)),
            full((HIDDEN, HIDDEN)), full((1, HIDDEN)),
            full((N_TGT, HIDDEN, CLS_H)), full((N_TGT, 1, CLS_H)),
            full((N_TGT, CLS_H, NUM_CLASSES)), full((N_TGT, 1, NUM_CLASSES)),
        ],
        out_specs=pl.BlockSpec((N_TGT, block_b, NUM_CLASSES), lambda i: (0, i, 0)),
        out_shape=jax.ShapeDtypeStruct((N_TGT, b_total, NUM_CLASSES), jnp.float32),
        compiler_params=pltpu.CompilerParams(
            dimension_semantics=("parallel",)),
    )(feats, w1, b1, w2, b2, c1w, c1b, c2w, c2b, hw1, hb1, hw2, hb2)
    return out


def kernel(var_0_raw, var_1_raw, var_2_raw, var_3_raw, var_4_raw, var_5_raw,
           var_6_raw, var_7_raw, var_8_raw, var_9_raw, var_10_raw, var_11_raw,
           var_12_raw, var_13_raw, var_14_raw, var_15_raw, params):
    feats = jnp.stack(
        (var_0_raw, var_1_raw, var_2_raw, var_3_raw, var_4_raw, var_5_raw,
         var_6_raw, var_7_raw, var_8_raw, var_9_raw, var_10_raw, var_11_raw,
         var_12_raw, var_13_raw, var_14_raw, var_15_raw), axis=0)
    p = params
    targets = [f"var_{i}" for i in range(N_TGT)]
    hw1 = jnp.stack([p[f"cls_{t}_W1"] for t in targets], axis=0)
    hb1 = jnp.stack([p[f"cls_{t}_b1"].reshape(1, CLS_H) for t in targets], axis=0)
    hw2 = jnp.stack([p[f"cls_{t}_W2"] for t in targets], axis=0)
    hb2 = jnp.stack([p[f"cls_{t}_b2"].reshape(1, NUM_CLASSES) for t in targets], axis=0)
    return _run(
        feats,
        p["enc_W1"], p["enc_b1"].reshape(1, HIDDEN),
        p["enc_W2"], p["enc_b2"].reshape(1, HIDDEN),
        p["conv1_W"], p["conv1_b"].reshape(1, HIDDEN),
        p["conv2_W"], p["conv2_b"].reshape(1, HIDDEN),
        hw1, hb1, hw2, hb2)
